# Initial kernel scaffold; baseline (speedup 1.0000x reference)
import jax
import jax.numpy as jnp
from jax.experimental import pallas as pl


def kernel(args, input, embed, labels, bts, ctx, eda, weights, W, b):
    raise NotImplementedError("write your pallas kernel here")


# speedup vs baseline: 2.3565x; 2.3565x over previous
import jax
import jax.numpy as jnp
from jax.experimental import pallas as pl
from jax.experimental.pallas import tpu as pltpu

TAU = 1.0
BM = 256


def _mm_kernel(idx_ref, x_ref, w_ref, b_ref, o_ref, wbf_ref):
    i = pl.program_id(0)

    @pl.when(i == 0)
    def _():
        wbf_ref[...] = w_ref[0].astype(jnp.bfloat16)

    x = x_ref[...].astype(jnp.bfloat16)
    acc = jax.lax.dot_general(
        x, wbf_ref[...], (((1,), (0,)), ((), ())),
        preferred_element_type=jnp.float32,
    )
    o_ref[...] = jnp.tanh(acc + b_ref[0, 0])


def kernel(args, input, embed, labels, bts, ctx, eda, weights, W, b):
    E, D, _ = W.shape
    Bb, S, _ = embed.shape
    M = Bb * S

    route_key = jax.random.fold_in(jax.random.key(0), 123)
    u = jax.random.uniform(route_key, weights.shape, minval=1e-6, maxval=1.0 - 1e-6)
    g = -jnp.log(-jnp.log(u))
    y_soft = jax.nn.softmax((weights + g) / TAU)
    idx = jnp.argmax(y_soft).astype(jnp.int32).reshape((1,))

    x2d = embed.reshape(M, D)
    b3 = b.reshape(E, 1, D)

    grid_spec = pltpu.PrefetchScalarGridSpec(
        num_scalar_prefetch=1,
        grid=(M // BM,),
        in_specs=[
            pl.BlockSpec((BM, D), lambda i, idx: (i, 0)),
            pl.BlockSpec((1, D, D), lambda i, idx: (idx[0], 0, 0)),
            pl.BlockSpec((1, 1, D), lambda i, idx: (idx[0], 0, 0)),
        ],
        out_specs=pl.BlockSpec((BM, D), lambda i, idx: (i, 0)),
        scratch_shapes=[pltpu.VMEM((D, D), jnp.bfloat16)],
    )
    out = pl.pallas_call(
        _mm_kernel,
        grid_spec=grid_spec,
        out_shape=jax.ShapeDtypeStruct((M, D), jnp.float32),
    )(idx, x2d, W, b3)
    return (input, out.reshape(Bb, S, D))


# speedup vs baseline: 2.4536x; 1.0412x over previous
import jax
import jax.numpy as jnp
from jax.experimental import pallas as pl
from jax.experimental.pallas import tpu as pltpu

TAU = 1.0
BM = 512
SUB = 256


def _mm_kernel(idx_ref, x_ref, w_ref, b_ref, o_ref, wbf_ref):
    i = pl.program_id(0)

    @pl.when(i == 0)
    def _():
        wbf_ref[...] = w_ref[0].astype(jnp.bfloat16)

    for t in range(BM // SUB):
        rows = pl.ds(t * SUB, SUB)
        x = x_ref[rows, :].astype(jnp.bfloat16)
        acc = jax.lax.dot_general(
            x, wbf_ref[...], (((1,), (0,)), ((), ())),
            preferred_element_type=jnp.float32,
        )
        o_ref[rows, :] = jnp.tanh(acc + b_ref[0, 0])


def kernel(args, input, embed, labels, bts, ctx, eda, weights, W, b):
    E, D, _ = W.shape
    Bb, S, _ = embed.shape
    M = Bb * S

    route_key = jax.random.fold_in(jax.random.key(0), 123)
    u = jax.random.uniform(route_key, weights.shape, minval=1e-6, maxval=1.0 - 1e-6)
    g = -jnp.log(-jnp.log(u))
    y_soft = jax.nn.softmax((weights + g) / TAU)
    idx = jnp.argmax(y_soft).astype(jnp.int32).reshape((1,))

    x2d = embed.reshape(M, D)
    b3 = b.reshape(E, 1, D)

    grid_spec = pltpu.PrefetchScalarGridSpec(
        num_scalar_prefetch=1,
        grid=(M // BM,),
        in_specs=[
            pl.BlockSpec((BM, D), lambda i, idx: (i, 0)),
            pl.BlockSpec((1, D, D), lambda i, idx: (idx[0], 0, 0)),
            pl.BlockSpec((1, 1, D), lambda i, idx: (idx[0], 0, 0)),
        ],
        out_specs=pl.BlockSpec((BM, D), lambda i, idx: (i, 0)),
        scratch_shapes=[pltpu.VMEM((D, D), jnp.bfloat16)],
    )
    out = pl.pallas_call(
        _mm_kernel,
        grid_spec=grid_spec,
        out_shape=jax.ShapeDtypeStruct((M, D), jnp.float32),
        compiler_params=pltpu.CompilerParams(
            vmem_limit_bytes=100 * 1024 * 1024,
        ),
    )(idx, x2d, W, b3)
    return (input, out.reshape(Bb, S, D))


# speedup vs baseline: 2.4601x; 1.0026x over previous
import jax
import jax.numpy as jnp
from jax.experimental import pallas as pl
from jax.experimental.pallas import tpu as pltpu

TAU = 1.0
BM = 1024
SUB = 256


def _mm_kernel(idx_ref, x_ref, w_ref, b_ref, o_ref, wbf_ref):
    i = pl.program_id(0)

    @pl.when(i == 0)
    def _():
        wbf_ref[...] = w_ref[0].astype(jnp.bfloat16)

    for t in range(BM // SUB):
        rows = pl.ds(t * SUB, SUB)
        x = x_ref[rows, :].astype(jnp.bfloat16)
        acc = jax.lax.dot_general(
            x, wbf_ref[...], (((1,), (0,)), ((), ())),
            preferred_element_type=jnp.float32,
        )
        o_ref[rows, :] = jnp.tanh(acc + b_ref[0, 0])


def kernel(args, input, embed, labels, bts, ctx, eda, weights, W, b):
    E, D, _ = W.shape
    Bb, S, _ = embed.shape
    M = Bb * S

    route_key = jax.random.fold_in(jax.random.key(0), 123)
    u = jax.random.uniform(route_key, weights.shape, minval=1e-6, maxval=1.0 - 1e-6)
    g = -jnp.log(-jnp.log(u))
    y_soft = jax.nn.softmax((weights + g) / TAU)
    idx = jnp.argmax(y_soft).astype(jnp.int32).reshape((1,))

    x2d = embed.reshape(M, D)
    b3 = b.reshape(E, 1, D)

    grid_spec = pltpu.PrefetchScalarGridSpec(
        num_scalar_prefetch=1,
        grid=(M // BM,),
        in_specs=[
            pl.BlockSpec((BM, D), lambda i, idx: (i, 0)),
            pl.BlockSpec((1, D, D), lambda i, idx: (idx[0], 0, 0)),
            pl.BlockSpec((1, 1, D), lambda i, idx: (idx[0], 0, 0)),
        ],
        out_specs=pl.BlockSpec((BM, D), lambda i, idx: (i, 0)),
        scratch_shapes=[pltpu.VMEM((D, D), jnp.bfloat16)],
    )
    out = pl.pallas_call(
        _mm_kernel,
        grid_spec=grid_spec,
        out_shape=jax.ShapeDtypeStruct((M, D), jnp.float32),
        compiler_params=pltpu.CompilerParams(
            vmem_limit_bytes=100 * 1024 * 1024,
        ),
    )(idx, x2d, W, b3)
    return (input, out.reshape(Bb, S, D))


# speedup vs baseline: 2.5426x; 1.0335x over previous
import jax
import jax.numpy as jnp
from jax.experimental import pallas as pl
from jax.experimental.pallas import tpu as pltpu

TAU = 1.0
BM = 1024
SUB = 256


def _mm_kernel(idx_ref, x_ref, w_ref, b_ref, o_ref, wbf_ref):
    i = pl.program_id(0)
    K2 = w_ref.shape[2]

    @pl.when(i == 0)
    def _():
        wbf_ref[: K2, :] = w_ref[0, 0].astype(jnp.bfloat16)
        for t in range(BM // SUB):
            rows = pl.ds(t * SUB, SUB)
            x = x_ref[rows, :K2].astype(jnp.bfloat16)
            o_ref[rows, :] = jax.lax.dot_general(
                x, wbf_ref[: K2, :], (((1,), (0,)), ((), ())),
                preferred_element_type=jnp.float32,
            )

    @pl.when(i == 1)
    def _():
        wbf_ref[K2:, :] = w_ref[0, 0].astype(jnp.bfloat16)
        for t in range(BM // SUB):
            rows = pl.ds(t * SUB, SUB)
            x = x_ref[rows, K2:].astype(jnp.bfloat16)
            acc = jax.lax.dot_general(
                x, wbf_ref[K2:, :], (((1,), (0,)), ((), ())),
                preferred_element_type=jnp.float32,
            )
            o_ref[rows, :] = jnp.tanh(acc + o_ref[rows, :] + b_ref[0, 0])

    @pl.when(i >= 2)
    def _():
        for t in range(BM // SUB):
            rows = pl.ds(t * SUB, SUB)
            x = x_ref[rows, :].astype(jnp.bfloat16)
            acc = jax.lax.dot_general(
                x, wbf_ref[...], (((1,), (0,)), ((), ())),
                preferred_element_type=jnp.float32,
            )
            o_ref[rows, :] = jnp.tanh(acc + b_ref[0, 0])


def kernel(args, input, embed, labels, bts, ctx, eda, weights, W, b):
    E, D, _ = W.shape
    Bb, S, _ = embed.shape
    M = Bb * S
    K2 = D // 2

    route_key = jax.random.fold_in(jax.random.key(0), 123)
    u = jax.random.uniform(route_key, weights.shape, minval=1e-6, maxval=1.0 - 1e-6)
    g = -jnp.log(-jnp.log(u))
    y_soft = jax.nn.softmax((weights + g) / TAU)
    idx = jnp.argmax(y_soft).astype(jnp.int32).reshape((1,))

    x2d = embed.reshape(M, D)
    b3 = b.reshape(E, 1, D)
    W4 = W.reshape(E, 2, K2, D)

    grid_spec = pltpu.PrefetchScalarGridSpec(
        num_scalar_prefetch=1,
        grid=(M // BM + 1,),
        in_specs=[
            pl.BlockSpec((BM, D), lambda i, idx: (jnp.maximum(i - 1, 0), 0)),
            pl.BlockSpec((1, 1, K2, D), lambda i, idx: (idx[0], jnp.minimum(i, 1), 0, 0)),
            pl.BlockSpec((1, 1, D), lambda i, idx: (idx[0], 0, 0)),
        ],
        out_specs=pl.BlockSpec((BM, D), lambda i, idx: (jnp.maximum(i - 1, 0), 0)),
        scratch_shapes=[
            pltpu.VMEM((D, D), jnp.bfloat16),
        ],
    )
    out = pl.pallas_call(
        _mm_kernel,
        grid_spec=grid_spec,
        out_shape=jax.ShapeDtypeStruct((M, D), jnp.float32),
        compiler_params=pltpu.CompilerParams(
            vmem_limit_bytes=63 * 1024 * 1024,
        ),
    )(idx, x2d, W4, b3)
    return (input, out.reshape(Bb, S, D))
